# Initial kernel scaffold; baseline (speedup 1.0000x reference)
import jax
import jax.numpy as jnp
from jax.experimental import pallas as pl


def kernel(Z, Z_m, Z_t, X, Y, Y_m, mask, atom_mask, residue_idx, chain_labels, pos_W, pos_b, edge_W, gamma, beta):
    raise NotImplementedError("write your pallas kernel here")


# speedup vs baseline: 1.0867x; 1.0867x over previous
import functools

import jax
import jax.numpy as jnp
import numpy as np
from jax.experimental import pallas as pl

TOP_K = 30
NUM_RBF = 16
MAX_REL = 32
_R = 64

_INTERPRET = False

_A_IDX = (0, 2, 3, 4, 1, 1, 1, 1, 0, 0, 0, 4, 4, 3, 0, 2, 3, 4, 2, 3, 4, 2, 3, 2)
_B_IDX = (0, 2, 3, 4, 0, 2, 3, 4, 2, 3, 4, 2, 3, 2, 1, 1, 1, 1, 0, 0, 0, 4, 4, 3)

_HI = jax.lax.Precision.HIGHEST


def _selector_consts():
    sa = np.zeros((16, 72), np.float32)
    sb = np.zeros((16, 72), np.float32)
    ss = np.zeros((72, 24), np.float32)
    for p in range(24):
        for c in range(3):
            sa[_A_IDX[p] * 3 + c, p * 3 + c] = 1.0
            sb[_B_IDX[p] * 3 + c, p * 3 + c] = 1.0
            ss[p * 3 + c, p] = 1.0
    smu = np.zeros((26, 400), np.float32)
    mu = np.linspace(2.0, 22.0, NUM_RBF).astype(np.float32)
    for g in range(25):
        for k in range(NUM_RBF):
            smu[g, g * NUM_RBF + k] = 1.0
            smu[25, g * NUM_RBF + k] = -mu[k]
    return jnp.asarray(sa), jnp.asarray(sb), jnp.asarray(ss), jnp.asarray(smu)


def _body(x2r, xt, mrow, mr, rrow, rc, yx, yy, yz, zt, sa, sb, ss, smu,
          w1, w2, b1, gm, bt, e_out, eidx_out, cbj_out):
    L = xt.shape[2]
    R = x2r.shape[1]
    P = R * TOP_K

    xtf = xt[0]

    def atoms_from_rows(x2):
        n = x2[:, 0:3]
        ca = x2[:, 3:6]
        cc = x2[:, 6:9]
        oo = x2[:, 12:15]
        b = ca - n
        c = cc - ca
        ax = b[:, 1:2] * c[:, 2:3] - b[:, 2:3] * c[:, 1:2]
        ay = b[:, 2:3] * c[:, 0:1] - b[:, 0:1] * c[:, 2:3]
        az = b[:, 0:1] * c[:, 1:2] - b[:, 1:2] * c[:, 0:1]
        a = jnp.concatenate([ax, ay, az], axis=1)
        cb = -0.58273431 * a + 0.56802827 * b - 0.54067466 * c + ca
        return n, ca, cc, oo, cb

    nr, car, ccr, oor, cbr = atoms_from_rows(x2r[0])
    a_table = jnp.concatenate([nr, car, ccr, oor, cbr, rrow[0]], axis=1)

    def row(i):
        return xtf[i:i + 1, :]

    bx, by, bz = row(3) - row(0), row(4) - row(1), row(5) - row(2)
    cx, cy, cz = row(6) - row(3), row(7) - row(4), row(8) - row(5)
    axr = by * cz - bz * cy
    ayr = bz * cx - bx * cz
    azr = bx * cy - by * cx
    cbx = -0.58273431 * axr + 0.56802827 * bx - 0.54067466 * cx + row(3)
    cby = -0.58273431 * ayr + 0.56802827 * by - 0.54067466 * cy + row(4)
    cbz = -0.58273431 * azr + 0.56802827 * bz - 0.54067466 * cz + row(5)
    tcols = jnp.concatenate(
        [row(0), row(1), row(2),
         row(3), row(4), row(5),
         row(6), row(7), row(8),
         row(12), row(13), row(14),
         cbx, cby, cbz,
         jnp.transpose(rc[0])], axis=0)
    table = jnp.transpose(tcols)

    dx = car[:, 0:1] - row(3)
    dy = car[:, 1:2] - row(4)
    dz = car[:, 2:3] - row(5)
    d_full = jnp.sqrt(dx * dx + dy * dy + dz * dz + 1e-6)
    m2 = mrow[0] * mr[0, 0:1, :]
    dm = m2 * d_full
    dmax = jnp.max(dm, axis=1, keepdims=True)
    dadj = dm + (1.0 - m2) * dmax

    lane_l_f = jax.lax.broadcasted_iota(jnp.int32, (1, L), 1).astype(jnp.float32)
    lane32f = jax.lax.broadcasted_iota(jnp.int32, (R, 32), 1).astype(jnp.float32)
    vals = jnp.zeros((R, 32), jnp.float32)
    idxs = jnp.zeros((R, 32), jnp.float32)
    dw = dadj
    for t in range(TOP_K):
        m = jnp.min(dw, axis=1, keepdims=True)
        sel = jnp.where(dw == m, jnp.broadcast_to(lane_l_f, dw.shape), 1e9)
        idxf = jnp.min(sel, axis=1, keepdims=True)
        vals = jnp.where(lane32f == t, m, vals)
        idxs = jnp.where(lane32f == t, idxf, idxs)
        dw = jnp.where(lane_l_f == idxf, 1e30, dw)

    eidx_out[0] = (idxs + 0.5).astype(jnp.int32)

    prow = jax.lax.broadcasted_iota(jnp.int32, (P, R), 0)
    pcol = jax.lax.broadcasted_iota(jnp.int32, (P, R), 1)
    rep = (prow // TOP_K == pcol).astype(jnp.float32)
    msel_r = jax.lax.broadcasted_iota(jnp.int32, (P, 32), 0) % TOP_K
    msel_l = jax.lax.broadcasted_iota(jnp.int32, (P, 32), 1)
    msel = (msel_r == msel_l).astype(jnp.float32)
    b1v = jnp.dot(rep, vals, precision=_HI) * msel
    b1i = jnp.dot(rep, idxs, precision=_HI) * msel
    dnb = jnp.sum(b1v, axis=1, keepdims=True)
    eflat = jnp.sum(b1i, axis=1, keepdims=True)
    eidx_i = (eflat + 0.5).astype(jnp.int32)

    lane_li = jax.lax.broadcasted_iota(jnp.int32, (P, L), 1)
    onehot = (lane_li == eidx_i).astype(jnp.float32)
    g16 = jnp.dot(onehot, table, precision=_HI)
    a_exp = jnp.dot(rep, a_table, precision=_HI)

    pa = jnp.dot(a_exp, sa[...], precision=_HI)
    pg = jnp.dot(g16, sb[...], precision=_HI)
    diff = pa - pg
    d2 = jnp.dot(diff * diff, ss[...], precision=_HI)
    d24 = jnp.sqrt(d2 + 1e-6)

    ones_p = jnp.ones((P, 1), jnp.float32)
    dall = jnp.concatenate([dnb, d24, ones_p], axis=1)
    dc = jnp.dot(dall, smu[...], precision=_HI)
    z = dc * 0.8
    feats = jnp.exp(-(z * z))

    off = a_exp[:, 15:16] - g16[:, 15:16]
    dpos = jnp.clip(off + float(MAX_REL), 0.0, float(2 * MAX_REL))
    dpos_i = (dpos + 0.5).astype(jnp.int32)
    lane66 = jax.lax.broadcasted_iota(jnp.int32, (P, 2 * MAX_REL + 2), 1)
    oh66 = (lane66 == dpos_i).astype(jnp.float32)

    e_pre = (jnp.dot(feats, w2[...], precision=_HI)
             + jnp.dot(oh66, w1[...], precision=_HI) + b1[...])

    mu_e = jnp.mean(e_pre, axis=1, keepdims=True)
    xm = e_pre - mu_e
    var = jnp.mean(xm * xm, axis=1, keepdims=True)
    e_out[0] = xm / jnp.sqrt(var + 1e-5) * gm[...] + bt[...]

    cbx_r, cby_r, cbz_r = cbr[:, 0:1], cbr[:, 1:2], cbr[:, 2:3]
    d78 = ((cbx_r - yx[0]) ** 2 + (cby_r - yy[0]) ** 2
           + (cbz_r - yz[0]) ** 2)
    zx, zy, zz = zt[0, 0:1, :], zt[0, 1:2, :], zt[0, 2:3, :]
    d16 = (cbx_r - zx) ** 2 + (cby_r - zy) ** 2 + (cbz_r - zz) ** 2
    cbj_out[0] = jnp.sqrt(jnp.concatenate([d78, d16], axis=1) + 1e-6)


def kernel(Z, Z_m, Z_t, X, Y, Y_m, mask, atom_mask, residue_idx,
           chain_labels, pos_W, pos_b, edge_W, gamma, beta):
    B, L = X.shape[0], X.shape[1]
    R = _R
    nblk = L // R
    P = R * TOP_K

    x2 = X.reshape(B, L, 15)
    xt = jnp.transpose(x2, (0, 2, 1))
    mask_c = mask[..., None]
    mask_r = mask[:, None, :]
    resid_c = residue_idx.astype(jnp.float32)[..., None]
    yr = Y.reshape(B, L, 234)
    yx = yr[:, :, 0::3]
    yy = yr[:, :, 1::3]
    yz = yr[:, :, 2::3]
    ztr = jnp.transpose(Z, (0, 2, 1))

    sa, sb, ss, smu = _selector_consts()
    w1 = jnp.dot(pos_W, edge_W[:16], precision=_HI)
    b1 = jnp.dot(pos_b[None, :], edge_W[:16], precision=_HI)
    w2 = edge_W[16:]
    gm = gamma[None, :]
    bt = beta[None, :]

    full = lambda shape: pl.BlockSpec(shape, lambda b, r: (0,) * len(shape))
    per_b = lambda shape: pl.BlockSpec(shape, lambda b, r: (b,) + (0,) * (len(shape) - 1))
    per_br = lambda shape: pl.BlockSpec(shape, lambda b, r: (b, r) + (0,) * (len(shape) - 2))

    out_shapes = (
        jax.ShapeDtypeStruct((B, L * TOP_K, 128), jnp.float32),
        jax.ShapeDtypeStruct((B, L, 32), jnp.int32),
        jax.ShapeDtypeStruct((B, L, 94), jnp.float32),
    )
    out_specs = (per_br((1, P, 128)), per_br((1, R, 32)), per_br((1, R, 94)))

    in_specs = [
        per_br((1, R, 15)),
        per_b((1, 15, L)),
        per_br((1, R, 1)),
        per_b((1, 1, L)),
        per_br((1, R, 1)),
        per_b((1, L, 1)),
        per_br((1, R, 78)),
        per_br((1, R, 78)),
        per_br((1, R, 78)),
        per_b((1, 3, 16)),
        full((16, 72)),
        full((16, 72)),
        full((72, 24)),
        full((26, 400)),
        full((66, 128)),
        full((400, 128)),
        full((1, 128)),
        full((1, 128)),
        full((1, 128)),
    ]

    e_full, eidx, cbj = pl.pallas_call(
        _body,
        grid=(B, nblk),
        in_specs=in_specs,
        out_specs=out_specs,
        out_shape=out_shapes,
        interpret=_INTERPRET,
    )(x2, xt, mask_c, mask_r, resid_c, resid_c, yx, yy, yz, ztr,
      sa, sb, ss, smu, w1, w2, b1, gm, bt)

    E = e_full.reshape(B, L, TOP_K, 128)
    E_idx = eidx[:, :, :TOP_K]
    return E, E_idx, cbj


# speedup vs baseline: 1.4477x; 1.3322x over previous
import functools

import jax
import jax.numpy as jnp
import numpy as np
from jax.experimental import pallas as pl

TOP_K = 30
NUM_RBF = 16
MAX_REL = 32
_R = 64

_INTERPRET = False

_A_IDX = (0, 2, 3, 4, 1, 1, 1, 1, 0, 0, 0, 4, 4, 3, 0, 2, 3, 4, 2, 3, 4, 2, 3, 2)
_B_IDX = (0, 2, 3, 4, 0, 2, 3, 4, 2, 3, 4, 2, 3, 2, 1, 1, 1, 1, 0, 0, 0, 4, 4, 3)

_HI = jax.lax.Precision.HIGHEST


def _split(x):
    hi = x.astype(jnp.bfloat16)
    lo = (x - hi.astype(jnp.float32)).astype(jnp.bfloat16)
    return hi, lo


def _mm_lhs01(a, b):
    ah = a.astype(jnp.bfloat16)
    bh, bl = _split(b)
    f32 = jnp.float32
    return (jnp.dot(ah, bh, preferred_element_type=f32)
            + jnp.dot(ah, bl, preferred_element_type=f32))


def _mm_rhs01(a, b):
    ah, al = _split(a)
    bh = b.astype(jnp.bfloat16)
    f32 = jnp.float32
    return (jnp.dot(ah, bh, preferred_element_type=f32)
            + jnp.dot(al, bh, preferred_element_type=f32))


def _mm3(a, b):
    ah, al = _split(a)
    bh, bl = _split(b)
    f32 = jnp.float32
    return (jnp.dot(ah, bh, preferred_element_type=f32)
            + jnp.dot(ah, bl, preferred_element_type=f32)
            + jnp.dot(al, bh, preferred_element_type=f32))


def _selector_consts():
    sa = np.zeros((16, 73), np.float32)
    sb = np.zeros((16, 73), np.float32)
    ss = np.zeros((72, 24), np.float32)
    sa[15, 72] = 1.0
    sb[15, 72] = 1.0
    for p in range(24):
        for c in range(3):
            sa[_A_IDX[p] * 3 + c, p * 3 + c] = 1.0
            sb[_B_IDX[p] * 3 + c, p * 3 + c] = 1.0
            ss[p * 3 + c, p] = 1.0
    smu = np.zeros((26, 400), np.float32)
    mu = np.linspace(2.0, 22.0, NUM_RBF).astype(np.float32)
    for g in range(25):
        for k in range(NUM_RBF):
            smu[g, g * NUM_RBF + k] = 1.0
            smu[25, g * NUM_RBF + k] = -mu[k]
    return jnp.asarray(sa), jnp.asarray(sb), jnp.asarray(ss), jnp.asarray(smu)


def _body(x2r, xt, mrow, mr, rrow, rc, yx, yy, yz, zt, sa, sb, ss, smu,
          w1, w2, b1, gm, bt, e_out, eidx_out, cbj_out):
    L = xt.shape[2]
    R = x2r.shape[1]
    P = R * TOP_K

    xtf = xt[0]

    def atoms_from_rows(x2):
        n = x2[:, 0:3]
        ca = x2[:, 3:6]
        cc = x2[:, 6:9]
        oo = x2[:, 12:15]
        b = ca - n
        c = cc - ca
        ax = b[:, 1:2] * c[:, 2:3] - b[:, 2:3] * c[:, 1:2]
        ay = b[:, 2:3] * c[:, 0:1] - b[:, 0:1] * c[:, 2:3]
        az = b[:, 0:1] * c[:, 1:2] - b[:, 1:2] * c[:, 0:1]
        a = jnp.concatenate([ax, ay, az], axis=1)
        cb = -0.58273431 * a + 0.56802827 * b - 0.54067466 * c + ca
        return n, ca, cc, oo, cb

    nr, car, ccr, oor, cbr = atoms_from_rows(x2r[0])
    a_table = jnp.concatenate([nr, car, ccr, oor, cbr, rrow[0]], axis=1)

    def row(i):
        return xtf[i:i + 1, :]

    bx, by, bz = row(3) - row(0), row(4) - row(1), row(5) - row(2)
    cx, cy, cz = row(6) - row(3), row(7) - row(4), row(8) - row(5)
    axr = by * cz - bz * cy
    ayr = bz * cx - bx * cz
    azr = bx * cy - by * cx
    cbx = -0.58273431 * axr + 0.56802827 * bx - 0.54067466 * cx + row(3)
    cby = -0.58273431 * ayr + 0.56802827 * by - 0.54067466 * cy + row(4)
    cbz = -0.58273431 * azr + 0.56802827 * bz - 0.54067466 * cz + row(5)
    tcols = jnp.concatenate(
        [row(0), row(1), row(2),
         row(3), row(4), row(5),
         row(6), row(7), row(8),
         row(12), row(13), row(14),
         cbx, cby, cbz,
         jnp.transpose(rc[0])], axis=0)
    table = jnp.transpose(tcols)

    dx = car[:, 0:1] - row(3)
    dy = car[:, 1:2] - row(4)
    dz = car[:, 2:3] - row(5)
    d_full = jnp.sqrt(dx * dx + dy * dy + dz * dz + 1e-6)
    m2 = mrow[0] * mr[0, 0:1, :]
    dm = m2 * d_full
    dmax = jnp.max(dm, axis=1, keepdims=True)
    dadj = dm + (1.0 - m2) * dmax

    lane_l_f = jax.lax.broadcasted_iota(jnp.int32, (1, L), 1).astype(jnp.float32)
    lane32f = jax.lax.broadcasted_iota(jnp.int32, (R, 32), 1).astype(jnp.float32)
    vals = jnp.zeros((R, 32), jnp.float32)
    idxs = jnp.zeros((R, 32), jnp.float32)
    dw = dadj
    for t in range(TOP_K):
        m = jnp.min(dw, axis=1, keepdims=True)
        sel = jnp.where(dw == m, jnp.broadcast_to(lane_l_f, dw.shape), 1e9)
        idxf = jnp.min(sel, axis=1, keepdims=True)
        vals = jnp.where(lane32f == t, m, vals)
        idxs = jnp.where(lane32f == t, idxf, idxs)
        dw = jnp.where(lane_l_f == idxf, 1e30, dw)

    eidx_out[0] = (idxs + 0.5).astype(jnp.int32)

    prow = jax.lax.broadcasted_iota(jnp.int32, (P, R), 0)
    pcol = jax.lax.broadcasted_iota(jnp.int32, (P, R), 1)
    rep = (prow // TOP_K == pcol).astype(jnp.float32)
    msel_r = jax.lax.broadcasted_iota(jnp.int32, (P, 32), 0) % TOP_K
    msel_l = jax.lax.broadcasted_iota(jnp.int32, (P, 32), 1)
    msel = (msel_r == msel_l).astype(jnp.float32)
    a_pair = _mm_rhs01(a_table, sa[...])
    cat = jnp.concatenate([vals, idxs, a_pair], axis=1)
    big = _mm_lhs01(rep, cat)
    dnb = jnp.sum(big[:, 0:32] * msel, axis=1, keepdims=True)
    eflat = jnp.sum(big[:, 32:64] * msel, axis=1, keepdims=True)
    eidx_i = (eflat + 0.5).astype(jnp.int32)
    pa = big[:, 64:136]
    resid_i = big[:, 136:137]

    lane_li = jax.lax.broadcasted_iota(jnp.int32, (P, L), 1)
    onehot = (lane_li == eidx_i).astype(jnp.float32)
    tb = _mm_rhs01(table, sb[...])
    g = _mm_lhs01(onehot, tb)
    pg = g[:, 0:72]
    resid_g = g[:, 72:73]

    diff = pa - pg
    d2 = _mm_rhs01(diff * diff, ss[...])
    d24 = jnp.sqrt(d2 + 1e-6)

    ones_p = jnp.ones((P, 1), jnp.float32)
    dall = jnp.concatenate([dnb, d24, ones_p], axis=1)
    dc = _mm3(dall, smu[...])
    z = dc * 0.8
    feats = jnp.exp(-(z * z))

    off = resid_i - resid_g
    dpos = jnp.clip(off + float(MAX_REL), 0.0, float(2 * MAX_REL))
    dpos_i = (dpos + 0.5).astype(jnp.int32)
    lane66 = jax.lax.broadcasted_iota(jnp.int32, (P, 2 * MAX_REL + 2), 1)
    oh66 = (lane66 == dpos_i).astype(jnp.float32)

    e_pre = (_mm3(feats, w2[...])
             + _mm_lhs01(oh66, w1[...]) + b1[...])

    mu_e = jnp.mean(e_pre, axis=1, keepdims=True)
    xm = e_pre - mu_e
    var = jnp.mean(xm * xm, axis=1, keepdims=True)
    e_out[0] = xm / jnp.sqrt(var + 1e-5) * gm[...] + bt[...]

    cbx_r, cby_r, cbz_r = cbr[:, 0:1], cbr[:, 1:2], cbr[:, 2:3]
    d78 = ((cbx_r - yx[0]) ** 2 + (cby_r - yy[0]) ** 2
           + (cbz_r - yz[0]) ** 2)
    zx, zy, zz = zt[0, 0:1, :], zt[0, 1:2, :], zt[0, 2:3, :]
    d16 = (cbx_r - zx) ** 2 + (cby_r - zy) ** 2 + (cbz_r - zz) ** 2
    cbj_out[0] = jnp.sqrt(jnp.concatenate([d78, d16], axis=1) + 1e-6)


def kernel(Z, Z_m, Z_t, X, Y, Y_m, mask, atom_mask, residue_idx,
           chain_labels, pos_W, pos_b, edge_W, gamma, beta):
    B, L = X.shape[0], X.shape[1]
    R = _R
    nblk = L // R
    P = R * TOP_K

    x2 = X.reshape(B, L, 15)
    xt = jnp.transpose(x2, (0, 2, 1))
    mask_c = mask[..., None]
    mask_r = mask[:, None, :]
    resid_c = residue_idx.astype(jnp.float32)[..., None]
    yr = Y.reshape(B, L, 234)
    yx = yr[:, :, 0::3]
    yy = yr[:, :, 1::3]
    yz = yr[:, :, 2::3]
    ztr = jnp.transpose(Z, (0, 2, 1))

    sa, sb, ss, smu = _selector_consts()
    w1 = jnp.dot(pos_W, edge_W[:16], precision=_HI)
    b1 = jnp.dot(pos_b[None, :], edge_W[:16], precision=_HI)
    w2 = edge_W[16:]
    gm = gamma[None, :]
    bt = beta[None, :]

    full = lambda shape: pl.BlockSpec(shape, lambda b, r: (0,) * len(shape))
    per_b = lambda shape: pl.BlockSpec(shape, lambda b, r: (b,) + (0,) * (len(shape) - 1))
    per_br = lambda shape: pl.BlockSpec(shape, lambda b, r: (b, r) + (0,) * (len(shape) - 2))

    out_shapes = (
        jax.ShapeDtypeStruct((B, L * TOP_K, 128), jnp.float32),
        jax.ShapeDtypeStruct((B, L, 32), jnp.int32),
        jax.ShapeDtypeStruct((B, L, 94), jnp.float32),
    )
    out_specs = (per_br((1, P, 128)), per_br((1, R, 32)), per_br((1, R, 94)))

    in_specs = [
        per_br((1, R, 15)),
        per_b((1, 15, L)),
        per_br((1, R, 1)),
        per_b((1, 1, L)),
        per_br((1, R, 1)),
        per_b((1, L, 1)),
        per_br((1, R, 78)),
        per_br((1, R, 78)),
        per_br((1, R, 78)),
        per_b((1, 3, 16)),
        full((16, 73)),
        full((16, 73)),
        full((72, 24)),
        full((26, 400)),
        full((66, 128)),
        full((400, 128)),
        full((1, 128)),
        full((1, 128)),
        full((1, 128)),
    ]

    e_full, eidx, cbj = pl.pallas_call(
        _body,
        grid=(B, nblk),
        in_specs=in_specs,
        out_specs=out_specs,
        out_shape=out_shapes,
        interpret=_INTERPRET,
    )(x2, xt, mask_c, mask_r, resid_c, resid_c, yx, yy, yz, ztr,
      sa, sb, ss, smu, w1, w2, b1, gm, bt)

    E = e_full.reshape(B, L, TOP_K, 128)
    E_idx = eidx[:, :, :TOP_K]
    return E, E_idx, cbj
